# baseline (speedup 1.0000x reference)
import functools

import jax
import jax.numpy as jnp
from jax import lax
from jax.experimental import pallas as pl
from jax.experimental.pallas import tpu as pltpu
from jax.experimental.pallas import tpu_sc as plsc

N = 50000
H = 64
HH = 32
E = 800000
GROUPS = 396
EPAD = 16 * GROUPS * 128
GB = 3
EB = GB * 128
NGRP = GROUPS // GB
NPAIR = NGRP // 2
DUMMY = N
ACC_ROWS = 50176
STRIPE = ACC_ROWS // 16
FCH = 784
NC_PAD = 50176
CSTRIPE = NC_PAD // 16
BR = 2000
NB = N // BR

_f32 = jnp.float32


def _emb_body(pf_ref, wp_ref, bp_ref, tf_ref, wt_ref, bt_ref, ph_ref, th_ref):
    ph_ref[...] = pf_ref[...] * wp_ref[...] + bp_ref[...]
    th_ref[...] = jnp.dot(tf_ref[...], wt_ref[...],
                          preferred_element_type=_f32) + bt_ref[...]


def _build_emb(interpret=False):
    return pl.pallas_call(
        _emb_body,
        grid=(NB,),
        in_specs=[
            pl.BlockSpec((BR, 1), lambda i: (i, 0)),
            pl.BlockSpec((1, H), lambda i: (0, 0)),
            pl.BlockSpec((1, H), lambda i: (0, 0)),
            pl.BlockSpec((BR, 8), lambda i: (i, 0)),
            pl.BlockSpec((8, H), lambda i: (0, 0)),
            pl.BlockSpec((1, H), lambda i: (0, 0)),
        ],
        out_specs=[
            pl.BlockSpec((BR, H), lambda i: (i, 0)),
            pl.BlockSpec((BR, H), lambda i: (i, 0)),
        ],
        out_shape=[
            jax.ShapeDtypeStruct((N, H), _f32),
            jax.ShapeDtypeStruct((N, H), _f32),
        ],
        interpret=interpret,
    )


def _stats_body(h_ref, w_ref, b_ref, wa_ref, c_ref, mz_ref, acc_ref):
    i = pl.program_id(0)
    m = jnp.dot(h_ref[...], w_ref[...], preferred_element_type=_f32) + b_ref[...]
    l = jnp.sum(m * wa_ref[...], axis=1, keepdims=True)
    c = c_ref[...]
    bmax = jnp.max(jnp.where(c > 0, l, -1e30))

    @pl.when(i == 0)
    def _():
        acc_ref[0] = -1e30
        acc_ref[1] = 0.0

    m_old = acc_ref[0]
    s_old = acc_ref[1]
    m_new = jnp.maximum(m_old, bmax)
    scale = jnp.exp(m_old - m_new)
    contrib = jnp.sum(jnp.where(c > 0, c * jnp.exp(l - m_new), 0.0))
    s_new = s_old * scale + contrib
    acc_ref[0] = m_new
    acc_ref[1] = s_new

    @pl.when(i == NB - 1)
    def _():
        mz_ref[0] = m_new
        mz_ref[1] = s_new


def _build_stats(interpret=False):
    return pl.pallas_call(
        _stats_body,
        grid=(NB,),
        in_specs=[
            pl.BlockSpec((BR, H), lambda i: (i, 0)),
            pl.BlockSpec((H, H), lambda i: (0, 0)),
            pl.BlockSpec((1, H), lambda i: (0, 0)),
            pl.BlockSpec((1, H), lambda i: (0, 0)),
            pl.BlockSpec((BR, 1), lambda i: (i, 0)),
        ],
        out_specs=pl.BlockSpec(memory_space=pltpu.SMEM),
        out_shape=jax.ShapeDtypeStruct((2,), _f32),
        scratch_shapes=[pltpu.SMEM((2,), _f32)],
        interpret=interpret,
    )


def _u_body(h_ref, w_ref, b_ref, wa_ref, mz_ref, u_ref):
    m = jnp.dot(h_ref[...], w_ref[...], preferred_element_type=_f32) + b_ref[...]
    l = jnp.sum(m * wa_ref[...], axis=1, keepdims=True)
    a = jnp.exp(l - mz_ref[0]) / mz_ref[1]
    u_ref[...] = (m * a).astype(jnp.bfloat16)


def _build_u(interpret=False):
    return pl.pallas_call(
        _u_body,
        grid=(NB,),
        in_specs=[
            pl.BlockSpec((BR, H), lambda i: (i, 0)),
            pl.BlockSpec((H, H), lambda i: (0, 0)),
            pl.BlockSpec((1, H), lambda i: (0, 0)),
            pl.BlockSpec((1, H), lambda i: (0, 0)),
            pl.BlockSpec(memory_space=pltpu.SMEM),
        ],
        out_specs=pl.BlockSpec((BR, H), lambda i: (i, 0)),
        out_shape=jax.ShapeDtypeStruct((N, H), jnp.bfloat16),
        interpret=interpret,
    )


def _upd_body(h_ref, mm_ref, w_ref, b_ref, out_ref, cs_ref, acc_ref):
    i = pl.program_id(0)
    h = h_ref[...]
    x = (jnp.dot(h, w_ref[0:H, :], preferred_element_type=_f32)
         + jnp.dot(mm_ref[...].astype(_f32), w_ref[H:2 * H, :],
                   preferred_element_type=_f32)
         + b_ref[...])
    hn = jnp.maximum(h + x, 0.0)
    out_ref[...] = hn

    @pl.when(i == 0)
    def _():
        acc_ref[...] = jnp.zeros((1, H), _f32)

    acc_ref[...] += jnp.sum(hn, axis=0, keepdims=True)

    @pl.when(i == NB - 1)
    def _():
        cs_ref[...] = acc_ref[...]


def _build_upd(interpret=False):
    return pl.pallas_call(
        _upd_body,
        grid=(NB,),
        in_specs=[
            pl.BlockSpec((BR, H), lambda i: (i, 0)),
            pl.BlockSpec((BR, H), lambda i: (i, 0)),
            pl.BlockSpec((2 * H, H), lambda i: (0, 0)),
            pl.BlockSpec((1, H), lambda i: (0, 0)),
        ],
        out_specs=[
            pl.BlockSpec((BR, H), lambda i: (i, 0)),
            pl.BlockSpec((1, H), lambda i: (0, 0)),
        ],
        out_shape=[
            jax.ShapeDtypeStruct((N, H), _f32),
            jax.ShapeDtypeStruct((1, H), _f32),
        ],
        scratch_shapes=[pltpu.VMEM((1, H), _f32)],
        interpret=interpret,
    )


def _head_body(csp_ref, cst_ref, px_ref,
               pew_ref, peb_ref, ppw_ref, ppb_ref, tpw_ref, tpb_ref,
               p1w_ref, p1b_ref, p2w_ref, p2b_ref,
               c1w_ref, c1b_ref, c2w_ref, c2b_ref, c3w_ref, c3b_ref,
               hp_ref, cf_ref):
    inv = 1.0 / N
    pg = jnp.dot(csp_ref[...] * inv, ppw_ref[...], preferred_element_type=_f32) + ppb_ref[...]
    tg = jnp.dot(cst_ref[...] * inv, tpw_ref[...], preferred_element_type=_f32) + tpb_ref[...]
    ph = jnp.dot(px_ref[...], pew_ref[...], preferred_element_type=_f32) + peb_ref[...]
    comb = jnp.concatenate([pg, tg, ph], axis=1)
    h1 = jnp.maximum(jnp.dot(comb, p1w_ref[...], preferred_element_type=_f32) + p1b_ref[...], 0.0)
    h2 = jnp.maximum(jnp.dot(h1, p2w_ref[...], preferred_element_type=_f32) + p2b_ref[...], 0.0)
    hp_ref[...] = h2
    c1 = jnp.maximum(jnp.dot(comb, c1w_ref[...], preferred_element_type=_f32) + c1b_ref[...], 0.0)
    c2 = jnp.maximum(jnp.dot(c1, c2w_ref[...], preferred_element_type=_f32) + c2b_ref[...], 0.0)
    z = jnp.sum(c2 * c3w_ref[...], axis=1, keepdims=True) + c3b_ref[0]
    cf_ref[...] = 1.0 / (1.0 + jnp.exp(-z))


def _build_head(interpret=False):
    return pl.pallas_call(
        _head_body,
        grid=(1,),
        in_specs=[pl.BlockSpec((1, H), lambda i: (0, 0)),
                  pl.BlockSpec((1, H), lambda i: (0, 0)),
                  pl.BlockSpec((1, 18), lambda i: (0, 0)),
                  pl.BlockSpec((18, H), lambda i: (0, 0)),
                  pl.BlockSpec((1, H), lambda i: (0, 0)),
                  pl.BlockSpec((H, H), lambda i: (0, 0)),
                  pl.BlockSpec((1, H), lambda i: (0, 0)),
                  pl.BlockSpec((H, H), lambda i: (0, 0)),
                  pl.BlockSpec((1, H), lambda i: (0, 0)),
                  pl.BlockSpec((3 * H, 2 * H), lambda i: (0, 0)),
                  pl.BlockSpec((1, 2 * H), lambda i: (0, 0)),
                  pl.BlockSpec((2 * H, H), lambda i: (0, 0)),
                  pl.BlockSpec((1, H), lambda i: (0, 0)),
                  pl.BlockSpec((3 * H, 2 * H), lambda i: (0, 0)),
                  pl.BlockSpec((1, 2 * H), lambda i: (0, 0)),
                  pl.BlockSpec((2 * H, H), lambda i: (0, 0)),
                  pl.BlockSpec((1, H), lambda i: (0, 0)),
                  pl.BlockSpec((1, H), lambda i: (0, 0)),
                  pl.BlockSpec(memory_space=pltpu.SMEM)],
        out_specs=[
            pl.BlockSpec((1, H), lambda i: (0, 0)),
            pl.BlockSpec((1, 1), lambda i: (0, 0)),
        ],
        out_shape=[
            jax.ShapeDtypeStruct((1, H), _f32),
            jax.ShapeDtypeStruct((1, 1), _f32),
        ],
        interpret=interpret,
    )


def _pred3_body(h_ref, w_ref, b_ref, out_ref):
    z = jnp.dot(h_ref[...], w_ref[...], preferred_element_type=_f32) + b_ref[...]
    out_ref[...] = 1.0 / (1.0 + jnp.exp(-z))


def _build_pred3(interpret=False):
    return pl.pallas_call(
        _pred3_body,
        grid=(1,),
        in_specs=[
            pl.BlockSpec((1, H), lambda i: (0, 0)),
            pl.BlockSpec((H, N), lambda i: (0, 0)),
            pl.BlockSpec((1, N), lambda i: (0, 0)),
        ],
        out_specs=pl.BlockSpec((1, N), lambda i: (0, 0)),
        out_shape=jax.ShapeDtypeStruct((1, N), _f32),
        interpret=interpret,
    )


def _sc_mesh():
    return plsc.VectorSubcoreMesh(core_axis_name="c", subcore_axis_name="s",
                                  num_cores=2, num_subcores=16)


_CCH = 2000
_NCCH = E // 16 // _CCH


def _counts_body(srcs_hbm, zeros_hbm, out_hbm, hist_v, idx_v, acc_v, tmp_v, shared):
    cid = lax.axis_index("c")
    sid = lax.axis_index("s")
    pltpu.sync_copy(zeros_hbm, hist_v)
    tile_base = cid * E + sid * (E // 16)
    ones = jnp.full((16,), 1.0, _f32)

    def chunk(ci, carry):
        pltpu.sync_copy(srcs_hbm.at[pl.ds(tile_base + ci * _CCH, _CCH)], idx_v)

        def inner(j, c2):
            iv = idx_v[pl.ds(j * 16, 16)]
            plsc.addupdate_scatter(hist_v, [iv], ones)
            return c2

        return lax.fori_loop(0, _CCH // 16, inner, carry)

    lax.fori_loop(0, _NCCH, chunk, 0)

    pltpu.sync_copy(hist_v, shared.at[pl.ds(sid * NC_PAD, NC_PAD)])
    plsc.subcore_barrier()

    sbase = sid * CSTRIPE
    pltpu.sync_copy(shared.at[pl.ds(sbase, CSTRIPE)], acc_v)
    for k in range(1, 16):
        pltpu.sync_copy(shared.at[pl.ds(k * NC_PAD + sbase, CSTRIPE)], tmp_v)

        def addj(j, c2):
            sl = pl.ds(j * 16, 16)
            acc_v[sl] = acc_v[sl] + tmp_v[sl]
            return c2

        lax.fori_loop(0, CSTRIPE // 16, addj, 0)
    pltpu.sync_copy(acc_v, out_hbm.at[pl.ds(cid * NC_PAD + sbase, CSTRIPE)])


def _build_counts(interpret=False):
    return functools.partial(
        pl.kernel,
        out_type=jax.ShapeDtypeStruct((2 * NC_PAD,), _f32),
        mesh=_sc_mesh(),
        scratch_types=[
            pltpu.VMEM((NC_PAD,), _f32),
            pltpu.VMEM((_CCH,), jnp.int32),
            pltpu.VMEM((CSTRIPE,), _f32),
            pltpu.VMEM((CSTRIPE,), _f32),
            pltpu.VMEM_SHARED((16 * NC_PAD,), _f32),
        ],
        compiler_params=pltpu.CompilerParams(needs_layout_passes=False),
        interpret=interpret,
    )(_counts_body)


def _edge_body(u_hbm, srcoff_hbm, dst_hbm, zeros_hbm, out_hbm,
               acc, srcA, srcB, dstA, dstB, rowsA, rowsB, conv,
               gsemA, gsemB, ssemA, ssemB):
    cid = lax.axis_index("c")
    sid = lax.axis_index("s")

    pltpu.sync_copy(zeros_hbm, acc.at[pl.ds(sid * STRIPE, STRIPE), :])
    plsc.subcore_barrier()

    base = cid * EPAD + sid * (GROUPS * 128)

    def load_idx(g, sref, dref):
        pltpu.sync_copy(srcoff_hbm.at[pl.ds(base + g * EB, EB)], sref)
        pltpu.sync_copy(dst_hbm.at[pl.ds(base + g * EB, EB)], dref)

    load_idx(0, srcA, dstA)
    pltpu.async_copy(u_hbm.at[srcA], rowsA, gsemA)

    def pair(k, carry):
        gA = 2 * k

        @pl.when(k > 0)
        def _():
            pltpu.make_async_copy(rowsB, acc.at[dstB], ssemB).wait()

        load_idx(gA + 1, srcB, dstB)
        pltpu.async_copy(u_hbm.at[srcB], rowsB, gsemB)
        pltpu.make_async_copy(u_hbm.at[srcA], rowsA, gsemA).wait()
        pltpu.async_copy(rowsA, acc.at[dstA], ssemA, add=True)

        @pl.when(k < NPAIR - 1)
        def _():
            pltpu.make_async_copy(rowsA, acc.at[dstA], ssemA).wait()
            load_idx(gA + 2, srcA, dstA)
            pltpu.async_copy(u_hbm.at[srcA], rowsA, gsemA)

        pltpu.make_async_copy(u_hbm.at[srcB], rowsB, gsemB).wait()
        pltpu.async_copy(rowsB, acc.at[dstB], ssemB, add=True)
        return carry

    lax.fori_loop(0, NPAIR, pair, 0)
    pltpu.make_async_copy(rowsA, acc.at[dstA], ssemA).wait()
    pltpu.make_async_copy(rowsB, acc.at[dstB], ssemB).wait()
    plsc.subcore_barrier()

    def flush_chunk(ci, carry):
        r0 = sid * STRIPE + ci * 64
        pltpu.sync_copy(acc.at[pl.ds(r0, 64), :], rowsA.at[pl.ds(0, 64), :])

        def cv(j, c2):
            r = j // 2
            cb = (j % 2) * 32
            x = rowsA[r, pl.ds(cb, 32)]
            xi = plsc.bitcast(x, jnp.int32)
            ev = plsc.bitcast(jnp.left_shift(xi, 16), _f32)
            od = plsc.bitcast(jnp.bitwise_and(xi, jnp.int32(-65536)), _f32)
            conv[r, pl.ds(cb // 2, 16)] = ev
            conv[r, pl.ds(32 + cb // 2, 16)] = od
            return c2

        lax.fori_loop(0, 128, cv, 0)
        pltpu.sync_copy(conv, out_hbm.at[pl.ds(cid * ACC_ROWS + r0, 64), :])
        return carry

    lax.fori_loop(0, STRIPE // 64, flush_chunk, 0)


def _build_edge(interpret=False):
    return functools.partial(
        pl.kernel,
        out_type=jax.ShapeDtypeStruct((2 * ACC_ROWS, H), _f32),
        mesh=_sc_mesh(),
        scratch_types=[
            pltpu.VMEM_SHARED((ACC_ROWS, H), jnp.bfloat16),
            pltpu.VMEM((EB,), jnp.int32),
            pltpu.VMEM((EB,), jnp.int32),
            pltpu.VMEM((EB,), jnp.int32),
            pltpu.VMEM((EB,), jnp.int32),
            pltpu.VMEM((EB, H), jnp.bfloat16),
            pltpu.VMEM((EB, H), jnp.bfloat16),
            pltpu.VMEM((64, H), _f32),
            pltpu.SemaphoreType.DMA,
            pltpu.SemaphoreType.DMA,
            pltpu.SemaphoreType.DMA,
            pltpu.SemaphoreType.DMA,
        ],
        compiler_params=pltpu.CompilerParams(needs_layout_passes=False,
                                             use_tc_tiling_on_sc=False),
        interpret=interpret,
    )(_edge_body)


_K_emb = _build_emb()
_K_stats = _build_stats()
_K_u = _build_u()
_K_upd = _build_upd()
_K_head = _build_head()
_K_pred3 = _build_pred3()


@functools.lru_cache(maxsize=None)
def _get_counts_kernel():
    return _build_counts()


@functools.lru_cache(maxsize=None)
def _get_edge_kernel():
    return _build_edge()


def _perm_upd_w(w):
    wm = w[H:2 * H]
    return jnp.concatenate([w[:H], wm[0::2], wm[1::2]])


def _prep_edges(pre_ei, post_ei):
    pad = EPAD - E
    zpad = jnp.zeros((pad,), jnp.int32)
    dpad = jnp.full((pad,), DUMMY, jnp.int32)
    src = jnp.concatenate([pre_ei[0], zpad, post_ei[0] + N, zpad])
    dst = jnp.concatenate([pre_ei[1], dpad, post_ei[1], dpad])
    return src, dst


def kernel(place_features, transition_features, prefix_encoding,
           pre_edge_index, post_edge_index, params):
    p = params
    row = lambda v: v.reshape(1, -1)

    srcs_flat = jnp.concatenate([pre_edge_index[0], post_edge_index[0]])
    zeros_n = jnp.zeros((NC_PAD,), _f32)
    counts = _get_counts_kernel()(srcs_flat, zeros_n).reshape(2, NC_PAD)
    c_pre = counts[0, :N].reshape(N, 1)
    c_post = counts[1, :N].reshape(N, 1)

    esrc, edst = _prep_edges(pre_edge_index, post_edge_index)
    zeros_b = jnp.zeros((STRIPE, H), jnp.bfloat16)

    ph, th = _K_emb(place_features, row(p['place_emb'][0][0]), row(p['place_emb'][1]),
                    transition_features, p['trans_emb'][0], row(p['trans_emb'][1]))

    for lp in p['layers']:
        mz_p = _K_stats(ph, lp['p2t'][0], row(lp['p2t'][1]), row(lp['t_att'][0][:, 0]), c_pre)
        u_p = _K_u(ph, lp['p2t'][0], row(lp['p2t'][1]), row(lp['t_att'][0][:, 0]), mz_p)
        mz_t = _K_stats(th, lp['t2p'][0], row(lp['t2p'][1]), row(lp['p_att'][0][:, 0]), c_post)
        u_t = _K_u(th, lp['t2p'][0], row(lp['t2p'][1]), row(lp['p_att'][0][:, 0]), mz_t)
        u_flat = jnp.concatenate([u_p, u_t], axis=0)
        msgs = _get_edge_kernel()(u_flat, esrc, edst, zeros_b).reshape(2, ACC_ROWS, H)

        ph, cs_p = _K_upd(ph, msgs[1], _perm_upd_w(lp['p_upd'][0]), row(lp['p_upd'][1]))
        th, cs_t = _K_upd(th, msgs[0], _perm_upd_w(lp['t_upd'][0]), row(lp['t_upd'][1]))

    hp, cf = _K_head(cs_p, cs_t, row(prefix_encoding),
                     p['prefix_emb'][0], row(p['prefix_emb'][1]),
                     p['place_pool'][0], row(p['place_pool'][1]),
                     p['trans_pool'][0], row(p['trans_pool'][1]),
                     p['pred1'][0], row(p['pred1'][1]),
                     p['pred2'][0], row(p['pred2'][1]),
                     p['conf1'][0], row(p['conf1'][1]),
                     p['conf2'][0], row(p['conf2'][1]),
                     row(p['conf3'][0][:, 0]), p['conf3'][1])

    nt = _K_pred3(hp, p['pred3'][0], row(p['pred3'][1]))
    return (nt.reshape(N), cf.reshape(1))


# speedup vs baseline: 1.0504x; 1.0504x over previous
import functools

import jax
import jax.numpy as jnp
from jax import lax
from jax.experimental import pallas as pl
from jax.experimental.pallas import tpu as pltpu
from jax.experimental.pallas import tpu_sc as plsc

N = 50000
H = 64
HH = 32
E = 800000
EPAD = 802816
GROUPS = 392
GB = 4
EB = GB * 128
NGRP = GROUPS // GB
DUMMY = N
ACC_ROWS = 50176
STRIPE = ACC_ROWS // 16
FCH = 784
NC_PAD = 50176
CSTRIPE = NC_PAD // 16
BR = 2000
NB = N // BR

_f32 = jnp.float32


def _emb_body(pf_ref, wp_ref, bp_ref, tf_ref, wt_ref, bt_ref, ph_ref, th_ref):
    ph_ref[...] = pf_ref[...] * wp_ref[...] + bp_ref[...]
    th_ref[...] = jnp.dot(tf_ref[...], wt_ref[...],
                          preferred_element_type=_f32) + bt_ref[...]


def _build_emb(interpret=False):
    return pl.pallas_call(
        _emb_body,
        grid=(NB,),
        in_specs=[
            pl.BlockSpec((BR, 1), lambda i: (i, 0)),
            pl.BlockSpec((1, H), lambda i: (0, 0)),
            pl.BlockSpec((1, H), lambda i: (0, 0)),
            pl.BlockSpec((BR, 8), lambda i: (i, 0)),
            pl.BlockSpec((8, H), lambda i: (0, 0)),
            pl.BlockSpec((1, H), lambda i: (0, 0)),
        ],
        out_specs=[
            pl.BlockSpec((BR, H), lambda i: (i, 0)),
            pl.BlockSpec((BR, H), lambda i: (i, 0)),
        ],
        out_shape=[
            jax.ShapeDtypeStruct((N, H), _f32),
            jax.ShapeDtypeStruct((N, H), _f32),
        ],
        interpret=interpret,
    )


def _stats_body(h_ref, w_ref, b_ref, wa_ref, c_ref, mz_ref, acc_ref):
    i = pl.program_id(0)
    m = jnp.dot(h_ref[...], w_ref[...], preferred_element_type=_f32) + b_ref[...]
    l = jnp.sum(m * wa_ref[...], axis=1, keepdims=True)
    c = c_ref[...]
    bmax = jnp.max(jnp.where(c > 0, l, -1e30))

    @pl.when(i == 0)
    def _():
        acc_ref[0] = -1e30
        acc_ref[1] = 0.0

    m_old = acc_ref[0]
    s_old = acc_ref[1]
    m_new = jnp.maximum(m_old, bmax)
    scale = jnp.exp(m_old - m_new)
    contrib = jnp.sum(jnp.where(c > 0, c * jnp.exp(l - m_new), 0.0))
    s_new = s_old * scale + contrib
    acc_ref[0] = m_new
    acc_ref[1] = s_new

    @pl.when(i == NB - 1)
    def _():
        mz_ref[0] = m_new
        mz_ref[1] = s_new


def _build_stats(interpret=False):
    return pl.pallas_call(
        _stats_body,
        grid=(NB,),
        in_specs=[
            pl.BlockSpec((BR, H), lambda i: (i, 0)),
            pl.BlockSpec((H, H), lambda i: (0, 0)),
            pl.BlockSpec((1, H), lambda i: (0, 0)),
            pl.BlockSpec((1, H), lambda i: (0, 0)),
            pl.BlockSpec((BR, 1), lambda i: (i, 0)),
        ],
        out_specs=pl.BlockSpec(memory_space=pltpu.SMEM),
        out_shape=jax.ShapeDtypeStruct((2,), _f32),
        scratch_shapes=[pltpu.SMEM((2,), _f32)],
        interpret=interpret,
    )


def _u_body(h_ref, w_ref, b_ref, wa_ref, mz_ref, u_ref):
    m = jnp.dot(h_ref[...], w_ref[...], preferred_element_type=_f32) + b_ref[...]
    l = jnp.sum(m * wa_ref[...], axis=1, keepdims=True)
    a = jnp.exp(l - mz_ref[0]) / mz_ref[1]
    u = m * a
    u_ref[0, :, :] = u[:, :HH]
    u_ref[1, :, :] = u[:, HH:]


def _build_u(interpret=False):
    return pl.pallas_call(
        _u_body,
        grid=(NB,),
        in_specs=[
            pl.BlockSpec((BR, H), lambda i: (i, 0)),
            pl.BlockSpec((H, H), lambda i: (0, 0)),
            pl.BlockSpec((1, H), lambda i: (0, 0)),
            pl.BlockSpec((1, H), lambda i: (0, 0)),
            pl.BlockSpec(memory_space=pltpu.SMEM),
        ],
        out_specs=pl.BlockSpec((2, BR, HH), lambda i: (0, i, 0)),
        out_shape=jax.ShapeDtypeStruct((2, N, HH), _f32),
        interpret=interpret,
    )


def _upd_body(h_ref, mm_ref, w_ref, b_ref, out_ref, cs_ref, acc_ref):
    i = pl.program_id(0)
    h = h_ref[...]
    x = (jnp.dot(h, w_ref[0:H, :], preferred_element_type=_f32)
         + jnp.dot(mm_ref[0, :, :], w_ref[H:H + HH, :], preferred_element_type=_f32)
         + jnp.dot(mm_ref[1, :, :], w_ref[H + HH:2 * H, :], preferred_element_type=_f32)
         + b_ref[...])
    hn = jnp.maximum(h + x, 0.0)
    out_ref[...] = hn

    @pl.when(i == 0)
    def _():
        acc_ref[...] = jnp.zeros((1, H), _f32)

    acc_ref[...] += jnp.sum(hn, axis=0, keepdims=True)

    @pl.when(i == NB - 1)
    def _():
        cs_ref[...] = acc_ref[...]


def _build_upd(interpret=False):
    return pl.pallas_call(
        _upd_body,
        grid=(NB,),
        in_specs=[
            pl.BlockSpec((BR, H), lambda i: (i, 0)),
            pl.BlockSpec((2, BR, HH), lambda i: (0, i, 0)),
            pl.BlockSpec((2 * H, H), lambda i: (0, 0)),
            pl.BlockSpec((1, H), lambda i: (0, 0)),
        ],
        out_specs=[
            pl.BlockSpec((BR, H), lambda i: (i, 0)),
            pl.BlockSpec((1, H), lambda i: (0, 0)),
        ],
        out_shape=[
            jax.ShapeDtypeStruct((N, H), _f32),
            jax.ShapeDtypeStruct((1, H), _f32),
        ],
        scratch_shapes=[pltpu.VMEM((1, H), _f32)],
        interpret=interpret,
    )


def _head_body(csp_ref, cst_ref, px_ref,
               pew_ref, peb_ref, ppw_ref, ppb_ref, tpw_ref, tpb_ref,
               p1w_ref, p1b_ref, p2w_ref, p2b_ref,
               c1w_ref, c1b_ref, c2w_ref, c2b_ref, c3w_ref, c3b_ref,
               hp_ref, cf_ref):
    inv = 1.0 / N
    pg = jnp.dot(csp_ref[...] * inv, ppw_ref[...], preferred_element_type=_f32) + ppb_ref[...]
    tg = jnp.dot(cst_ref[...] * inv, tpw_ref[...], preferred_element_type=_f32) + tpb_ref[...]
    ph = jnp.dot(px_ref[...], pew_ref[...], preferred_element_type=_f32) + peb_ref[...]
    comb = jnp.concatenate([pg, tg, ph], axis=1)
    h1 = jnp.maximum(jnp.dot(comb, p1w_ref[...], preferred_element_type=_f32) + p1b_ref[...], 0.0)
    h2 = jnp.maximum(jnp.dot(h1, p2w_ref[...], preferred_element_type=_f32) + p2b_ref[...], 0.0)
    hp_ref[...] = h2
    c1 = jnp.maximum(jnp.dot(comb, c1w_ref[...], preferred_element_type=_f32) + c1b_ref[...], 0.0)
    c2 = jnp.maximum(jnp.dot(c1, c2w_ref[...], preferred_element_type=_f32) + c2b_ref[...], 0.0)
    z = jnp.sum(c2 * c3w_ref[...], axis=1, keepdims=True) + c3b_ref[0]
    cf_ref[...] = 1.0 / (1.0 + jnp.exp(-z))


def _build_head(interpret=False):
    return pl.pallas_call(
        _head_body,
        grid=(1,),
        in_specs=[pl.BlockSpec((1, H), lambda i: (0, 0)),
                  pl.BlockSpec((1, H), lambda i: (0, 0)),
                  pl.BlockSpec((1, 18), lambda i: (0, 0)),
                  pl.BlockSpec((18, H), lambda i: (0, 0)),
                  pl.BlockSpec((1, H), lambda i: (0, 0)),
                  pl.BlockSpec((H, H), lambda i: (0, 0)),
                  pl.BlockSpec((1, H), lambda i: (0, 0)),
                  pl.BlockSpec((H, H), lambda i: (0, 0)),
                  pl.BlockSpec((1, H), lambda i: (0, 0)),
                  pl.BlockSpec((3 * H, 2 * H), lambda i: (0, 0)),
                  pl.BlockSpec((1, 2 * H), lambda i: (0, 0)),
                  pl.BlockSpec((2 * H, H), lambda i: (0, 0)),
                  pl.BlockSpec((1, H), lambda i: (0, 0)),
                  pl.BlockSpec((3 * H, 2 * H), lambda i: (0, 0)),
                  pl.BlockSpec((1, 2 * H), lambda i: (0, 0)),
                  pl.BlockSpec((2 * H, H), lambda i: (0, 0)),
                  pl.BlockSpec((1, H), lambda i: (0, 0)),
                  pl.BlockSpec((1, H), lambda i: (0, 0)),
                  pl.BlockSpec(memory_space=pltpu.SMEM)],
        out_specs=[
            pl.BlockSpec((1, H), lambda i: (0, 0)),
            pl.BlockSpec((1, 1), lambda i: (0, 0)),
        ],
        out_shape=[
            jax.ShapeDtypeStruct((1, H), _f32),
            jax.ShapeDtypeStruct((1, 1), _f32),
        ],
        interpret=interpret,
    )


def _pred3_body(h_ref, w_ref, b_ref, out_ref):
    z = jnp.dot(h_ref[...], w_ref[...], preferred_element_type=_f32) + b_ref[...]
    out_ref[...] = 1.0 / (1.0 + jnp.exp(-z))


def _build_pred3(interpret=False):
    return pl.pallas_call(
        _pred3_body,
        grid=(1,),
        in_specs=[
            pl.BlockSpec((1, H), lambda i: (0, 0)),
            pl.BlockSpec((H, N), lambda i: (0, 0)),
            pl.BlockSpec((1, N), lambda i: (0, 0)),
        ],
        out_specs=pl.BlockSpec((1, N), lambda i: (0, 0)),
        out_shape=jax.ShapeDtypeStruct((1, N), _f32),
        interpret=interpret,
    )


def _sc_mesh():
    return plsc.VectorSubcoreMesh(core_axis_name="c", subcore_axis_name="s",
                                  num_cores=2, num_subcores=16)


_CCH = 2000
_NCCH = E // 16 // _CCH


def _counts_body(srcs_hbm, zeros_hbm, out_hbm, hist_v, idx_v, acc_v, tmp_v, shared):
    cid = lax.axis_index("c")
    sid = lax.axis_index("s")
    pltpu.sync_copy(zeros_hbm, hist_v)
    tile_base = cid * E + sid * (E // 16)
    ones = jnp.full((16,), 1.0, _f32)

    def chunk(ci, carry):
        pltpu.sync_copy(srcs_hbm.at[pl.ds(tile_base + ci * _CCH, _CCH)], idx_v)

        def inner(j, c2):
            iv = idx_v[pl.ds(j * 16, 16)]
            plsc.addupdate_scatter(hist_v, [iv], ones)
            return c2

        return lax.fori_loop(0, _CCH // 16, inner, carry)

    lax.fori_loop(0, _NCCH, chunk, 0)

    pltpu.sync_copy(hist_v, shared.at[pl.ds(sid * NC_PAD, NC_PAD)])
    plsc.subcore_barrier()

    sbase = sid * CSTRIPE
    pltpu.sync_copy(shared.at[pl.ds(sbase, CSTRIPE)], acc_v)
    for k in range(1, 16):
        pltpu.sync_copy(shared.at[pl.ds(k * NC_PAD + sbase, CSTRIPE)], tmp_v)

        def addj(j, c2):
            sl = pl.ds(j * 16, 16)
            acc_v[sl] = acc_v[sl] + tmp_v[sl]
            return c2

        lax.fori_loop(0, CSTRIPE // 16, addj, 0)
    pltpu.sync_copy(acc_v, out_hbm.at[pl.ds(cid * NC_PAD + sbase, CSTRIPE)])


def _build_counts(interpret=False):
    return functools.partial(
        pl.kernel,
        out_type=jax.ShapeDtypeStruct((2 * NC_PAD,), _f32),
        mesh=_sc_mesh(),
        scratch_types=[
            pltpu.VMEM((NC_PAD,), _f32),
            pltpu.VMEM((_CCH,), jnp.int32),
            pltpu.VMEM((CSTRIPE,), _f32),
            pltpu.VMEM((CSTRIPE,), _f32),
            pltpu.VMEM_SHARED((16 * NC_PAD,), _f32),
        ],
        compiler_params=pltpu.CompilerParams(needs_layout_passes=False),
        interpret=interpret,
    )(_counts_body)


def _edge_body(u_hbm, srcoff_hbm, dst_hbm, zeros_hbm, out_hbm,
               acc, srcv, dstv, rows, gsem):
    cid = lax.axis_index("c")
    sid = lax.axis_index("s")

    pltpu.sync_copy(zeros_hbm, acc.at[pl.ds(sid * STRIPE, STRIPE), :])
    plsc.subcore_barrier()

    def group(gi, carry):
        e0 = cid * EPAD + sid * (GROUPS * 128) + gi * EB
        pltpu.sync_copy(srcoff_hbm.at[pl.ds(e0, EB)], srcv)
        e0d = sid * (GROUPS * 128) + gi * EB
        pltpu.sync_copy(dst_hbm.at[pl.ds(e0d, EB)], dstv)
        pltpu.async_copy(u_hbm.at[srcv], rows, gsem).wait()
        pltpu.sync_copy(rows, acc.at[dstv], add=True)
        return carry

    lax.fori_loop(0, NGRP, group, 0)
    plsc.subcore_barrier()

    r0 = sid * STRIPE
    pltpu.sync_copy(acc.at[pl.ds(r0, STRIPE), :],
                    out_hbm.at[pl.ds(cid * ACC_ROWS + r0, STRIPE), :])


def _build_edge(interpret=False):
    return functools.partial(
        pl.kernel,
        out_type=jax.ShapeDtypeStruct((2 * ACC_ROWS, HH), _f32),
        mesh=_sc_mesh(),
        scratch_types=[
            pltpu.VMEM_SHARED((ACC_ROWS, HH), _f32),
            pltpu.VMEM((EB,), jnp.int32),
            pltpu.VMEM((EB,), jnp.int32),
            pltpu.VMEM((EB, HH), _f32),
            pltpu.SemaphoreType.DMA,
        ],
        compiler_params=pltpu.CompilerParams(needs_layout_passes=False,
                                             use_tc_tiling_on_sc=False),
        interpret=interpret,
    )(_edge_body)


_K_emb = _build_emb()
_K_stats = _build_stats()
_K_u = _build_u()
_K_upd = _build_upd()
_K_head = _build_head()
_K_pred3 = _build_pred3()


@functools.lru_cache(maxsize=None)
def _get_counts_kernel():
    return _build_counts()


@functools.lru_cache(maxsize=None)
def _get_edge_kernel():
    return _build_edge()


def _prep_edges(ei):
    pad = EPAD - E
    src = jnp.concatenate([ei[0], jnp.zeros((pad,), jnp.int32)])
    dst = jnp.concatenate([ei[1], jnp.full((pad,), DUMMY, jnp.int32)])
    src_off = jnp.concatenate([src, src + N])
    return src_off, dst


def kernel(place_features, transition_features, prefix_encoding,
           pre_edge_index, post_edge_index, params):
    p = params
    row = lambda v: v.reshape(1, -1)

    srcs_flat = jnp.concatenate([pre_edge_index[0], post_edge_index[0]])
    zeros_n = jnp.zeros((NC_PAD,), _f32)
    counts = _get_counts_kernel()(srcs_flat, zeros_n).reshape(2, NC_PAD)
    c_pre = counts[0, :N].reshape(N, 1)
    c_post = counts[1, :N].reshape(N, 1)

    pre_src3, pre_dst3 = _prep_edges(pre_edge_index)
    post_src3, post_dst3 = _prep_edges(post_edge_index)
    zeros_b = jnp.zeros((STRIPE, HH), _f32)

    ph, th = _K_emb(place_features, row(p['place_emb'][0][0]), row(p['place_emb'][1]),
                    transition_features, p['trans_emb'][0], row(p['trans_emb'][1]))

    for lp in p['layers']:
        mz_p = _K_stats(ph, lp['p2t'][0], row(lp['p2t'][1]), row(lp['t_att'][0][:, 0]), c_pre)
        u_p = _K_u(ph, lp['p2t'][0], row(lp['p2t'][1]), row(lp['t_att'][0][:, 0]), mz_p)
        tm3 = _get_edge_kernel()(u_p.reshape(2 * N, HH), pre_src3, pre_dst3, zeros_b).reshape(2, ACC_ROWS, HH)

        mz_t = _K_stats(th, lp['t2p'][0], row(lp['t2p'][1]), row(lp['p_att'][0][:, 0]), c_post)
        u_t = _K_u(th, lp['t2p'][0], row(lp['t2p'][1]), row(lp['p_att'][0][:, 0]), mz_t)
        pm3 = _get_edge_kernel()(u_t.reshape(2 * N, HH), post_src3, post_dst3, zeros_b).reshape(2, ACC_ROWS, HH)

        ph, cs_p = _K_upd(ph, pm3, lp['p_upd'][0], row(lp['p_upd'][1]))
        th, cs_t = _K_upd(th, tm3, lp['t_upd'][0], row(lp['t_upd'][1]))

    hp, cf = _K_head(cs_p, cs_t, row(prefix_encoding),
                     p['prefix_emb'][0], row(p['prefix_emb'][1]),
                     p['place_pool'][0], row(p['place_pool'][1]),
                     p['trans_pool'][0], row(p['trans_pool'][1]),
                     p['pred1'][0], row(p['pred1'][1]),
                     p['pred2'][0], row(p['pred2'][1]),
                     p['conf1'][0], row(p['conf1'][1]),
                     p['conf2'][0], row(p['conf2'][1]),
                     row(p['conf3'][0][:, 0]), p['conf3'][1])

    nt = _K_pred3(hp, p['pred3'][0], row(p['pred3'][1]))
    return (nt.reshape(N), cf.reshape(1))
